# Initial kernel scaffold; baseline (speedup 1.0000x reference)
import jax
import jax.numpy as jnp
from jax.experimental import pallas as pl


def kernel(x, b_pre, W_enc, b_enc, W_dec, b_dec):
    raise NotImplementedError("write your pallas kernel here")


# speedup vs baseline: 6.0121x; 6.0121x over previous
import jax
import jax.numpy as jnp
from jax.experimental import pallas as pl
from jax.experimental.pallas import tpu as pltpu

N_TOK = 8192
D_MODEL = 2048
D_SAE = 16384
K = 64

RT = 256
LT = 512
NJ = D_SAE // LT
NI = N_TOK // RT

import numpy as np

_INT_MIN = np.int32(-2147483648)


def _key_to_f32(k):
    fbits = jnp.where(k < 0, k ^ _INT_MIN, ~k)
    return jax.lax.bitcast_convert_type(fbits, jnp.float32)


def _body(x_ref, bpre_ref, we_ref, benc_ref, wd_ref, bdec_ref,
          recon_ref, zs_ref, xs, zbuf, thr):
    j = pl.program_id(1)

    @pl.when(j == 0)
    def _():
        xs[...] = x_ref[...] - bpre_ref[...]

    @pl.when(j < NJ)
    def _():
        z = jax.lax.dot_general(
            xs[...], we_ref[...], (((1,), (1,)), ((), ())),
            preferred_element_type=jnp.float32)
        zbuf[:, pl.ds(pl.multiple_of(j * LT, LT), LT)] = z + benc_ref[...]

    @pl.when(j == NJ)
    def _():
        zb = zbuf[...]
        t = jnp.zeros((RT, 1), jnp.int32)
        for b in range(31, -1, -1):
            cand = t | np.int32((1 << b) - 4294967296 if b == 31 else 1 << b)
            cf = _key_to_f32(cand)
            cnt = jnp.sum((zb >= cf).astype(jnp.int32), axis=1, keepdims=True)
            t = jnp.where(cnt >= K, cand, t)
        thr[...] = _key_to_f32(t)

    @pl.when(j >= NJ)
    def _():
        jj = j - NJ
        zt = zbuf[:, pl.ds(pl.multiple_of(jj * LT, LT), LT)]
        zs = jnp.where(zt >= thr[...], zt, 0.0)
        zs_ref[...] = zs
        part = jax.lax.dot_general(
            zs, wd_ref[...], (((1,), (1,)), ((), ())),
            preferred_element_type=jnp.float32)

        @pl.when(j == NJ)
        def _():
            recon_ref[...] = part + bdec_ref[...]

        @pl.when(j > NJ)
        def _():
            recon_ref[...] += part


def _sae_call(x, b_pre2, W_enc, b_enc2, W_dec, b_dec2, interpret=False):
    return pl.pallas_call(
        _body,
        grid=(NI, 2 * NJ),
        in_specs=[
            pl.BlockSpec((RT, D_MODEL), lambda i, j: (i, 0)),
            pl.BlockSpec((1, D_MODEL), lambda i, j: (0, 0)),
            pl.BlockSpec((LT, D_MODEL), lambda i, j: (jnp.minimum(j, NJ - 1), 0)),
            pl.BlockSpec((1, LT), lambda i, j: (0, jnp.minimum(j, NJ - 1))),
            pl.BlockSpec((D_MODEL, LT), lambda i, j: (0, jnp.maximum(j - NJ, 0))),
            pl.BlockSpec((1, D_MODEL), lambda i, j: (0, 0)),
        ],
        out_specs=[
            pl.BlockSpec((RT, D_MODEL), lambda i, j: (i, 0)),
            pl.BlockSpec((RT, LT), lambda i, j: (i, jnp.maximum(j - NJ, 0))),
        ],
        out_shape=[
            jax.ShapeDtypeStruct((N_TOK, D_MODEL), jnp.float32),
            jax.ShapeDtypeStruct((N_TOK, D_SAE), jnp.float32),
        ],
        scratch_shapes=[
            pltpu.VMEM((RT, D_MODEL), jnp.float32),
            pltpu.VMEM((RT, D_SAE), jnp.float32),
            pltpu.VMEM((RT, 1), jnp.float32),
        ],
        compiler_params=pltpu.CompilerParams(
            dimension_semantics=("arbitrary", "arbitrary"),
        ),
        interpret=interpret,
    )(x, b_pre2, W_enc, b_enc2, W_dec, b_dec2)


def kernel(x, b_pre, W_enc, b_enc, W_dec, b_dec):
    recon, zs = _sae_call(
        x,
        b_pre.reshape(1, D_MODEL),
        W_enc,
        b_enc.reshape(1, D_SAE),
        W_dec,
        b_dec.reshape(1, D_MODEL),
    )
    return (recon, zs)


# speedup vs baseline: 6.1621x; 1.0249x over previous
import jax
import jax.numpy as jnp
from jax.experimental import pallas as pl
from jax.experimental.pallas import tpu as pltpu

N_TOK = 8192
D_MODEL = 2048
D_SAE = 16384
K = 64

RT = 256
LT = 512
NJ = D_SAE // LT
NI = N_TOK // RT

import numpy as np

_INT_MIN = np.int32(-2147483648)


def _key_to_f32(k):
    fbits = jnp.where(k < 0, k ^ _INT_MIN, ~k)
    return jax.lax.bitcast_convert_type(fbits, jnp.float32)


def _body(x_ref, bpre_ref, we_ref, benc_ref, wd_ref, bdec_ref,
          recon_ref, zs_ref, xs, zbuf, thr):
    j = pl.program_id(1)

    @pl.when(j == 0)
    def _():
        xs[...] = x_ref[...] - bpre_ref[...]

    @pl.when(j < NJ)
    def _():
        z = jax.lax.dot_general(
            xs[...], we_ref[...], (((1,), (1,)), ((), ())),
            preferred_element_type=jnp.float32)
        zbuf[:, pl.ds(pl.multiple_of(j * LT, LT), LT)] = z + benc_ref[...]

    @pl.when(j == NJ)
    def _():
        zb = zbuf[...]
        t = jnp.zeros((RT, 1), jnp.int32)
        for b in range(31, -1, -1):
            cand = t | np.int32((1 << b) - 4294967296 if b == 31 else 1 << b)
            cf = _key_to_f32(cand)
            cnt = jnp.sum((zb >= cf).astype(jnp.int32), axis=1, keepdims=True)
            t = jnp.where(cnt >= K, cand, t)
        thr[...] = _key_to_f32(t)

    @pl.when(j >= NJ)
    def _():
        jj = j - NJ
        zt = zbuf[:, pl.ds(pl.multiple_of(jj * LT, LT), LT)]
        zs = jnp.where(zt >= thr[...], zt, 0.0)
        zs_ref[...] = zs
        part = jax.lax.dot_general(
            zs.astype(jnp.bfloat16), wd_ref[...], (((1,), (1,)), ((), ())),
            preferred_element_type=jnp.float32)

        @pl.when(j == NJ)
        def _():
            recon_ref[...] = part + bdec_ref[...]

        @pl.when(j > NJ)
        def _():
            recon_ref[...] += part


def _sae_call(x, b_pre2, W_enc, b_enc2, W_dec, b_dec2, interpret=False):
    return pl.pallas_call(
        _body,
        grid=(NI, 2 * NJ),
        in_specs=[
            pl.BlockSpec((RT, D_MODEL), lambda i, j: (i, 0)),
            pl.BlockSpec((1, D_MODEL), lambda i, j: (0, 0)),
            pl.BlockSpec((LT, D_MODEL), lambda i, j: (jnp.minimum(j, NJ - 1), 0)),
            pl.BlockSpec((1, LT), lambda i, j: (0, jnp.minimum(j, NJ - 1))),
            pl.BlockSpec((D_MODEL, LT), lambda i, j: (0, jnp.maximum(j - NJ, 0))),
            pl.BlockSpec((1, D_MODEL), lambda i, j: (0, 0)),
        ],
        out_specs=[
            pl.BlockSpec((RT, D_MODEL), lambda i, j: (i, 0)),
            pl.BlockSpec((RT, LT), lambda i, j: (i, jnp.maximum(j - NJ, 0))),
        ],
        out_shape=[
            jax.ShapeDtypeStruct((N_TOK, D_MODEL), jnp.float32),
            jax.ShapeDtypeStruct((N_TOK, D_SAE), jnp.float32),
        ],
        scratch_shapes=[
            pltpu.VMEM((RT, D_MODEL), jnp.float32),
            pltpu.VMEM((RT, D_SAE), jnp.float32),
            pltpu.VMEM((RT, 1), jnp.float32),
        ],
        compiler_params=pltpu.CompilerParams(
            dimension_semantics=("arbitrary", "arbitrary"),
        ),
        interpret=interpret,
    )(x, b_pre2, W_enc, b_enc2, W_dec, b_dec2)


def kernel(x, b_pre, W_enc, b_enc, W_dec, b_dec):
    recon, zs = _sae_call(
        x,
        b_pre.reshape(1, D_MODEL),
        W_enc,
        b_enc.reshape(1, D_SAE),
        W_dec.astype(jnp.bfloat16),
        b_dec.reshape(1, D_MODEL),
    )
    return (recon, zs)


# speedup vs baseline: 8.6912x; 1.4104x over previous
import jax
import jax.numpy as jnp
import numpy as np
from jax.experimental import pallas as pl
from jax.experimental.pallas import tpu as pltpu

N_TOK = 8192
D_MODEL = 2048
D_SAE = 16384
K = 64

RT = 256
LT = 512
NJ = D_SAE // LT
NI = N_TOK // RT

_INT_MIN = np.int32(-2147483648)


def _key_to_f32(k):
    fbits = jnp.where(k < 0, k ^ _INT_MIN, ~k)
    return jax.lax.bitcast_convert_type(fbits, jnp.float32)


def _body(x_ref, bpre_ref, we_ref, benc_ref, wd_ref, bdec_ref,
          recon_ref, zs_ref, xs, zbuf0, zbuf1, tsel, thr):
    i = pl.program_id(0)
    j = pl.program_id(1)

    zb_cur = jnp.where(i % 2 == 0, 0, 1)

    @pl.when((i < NI) & (j == 0))
    def _():
        xs[...] = x_ref[...] - bpre_ref[...]

    @pl.when((i < NI) & (j < NJ))
    def _():
        z = jax.lax.dot_general(
            xs[...], we_ref[...], (((1,), (1,)), ((), ())),
            preferred_element_type=jnp.float32) + benc_ref[...]
        sl = pl.ds(pl.multiple_of(j * LT, LT), LT)

        @pl.when(zb_cur == 0)
        def _():
            zbuf0[:, sl] = z

        @pl.when(zb_cur == 1)
        def _():
            zbuf1[:, sl] = z

    def _sel_bits(zb):
        t = jnp.where(j == 0, jnp.zeros((RT, 1), jnp.int32), tsel[...])
        bit = jnp.left_shift(jnp.int32(1), 31 - j)
        cand = t | bit
        cf = _key_to_f32(cand)
        cnt = jnp.sum((zb >= cf).astype(jnp.int32), axis=1, keepdims=True)
        t = jnp.where(cnt >= K, cand, t)
        tsel[...] = t

        @pl.when(j == NJ - 1)
        def _():
            thr[...] = _key_to_f32(t)

    @pl.when((i >= 1) & (j < NJ) & (zb_cur == 1))
    def _():
        _sel_bits(zbuf0[...])

    @pl.when((i >= 1) & (j < NJ) & (zb_cur == 0))
    def _():
        _sel_bits(zbuf1[...])

    def _decode(zb_ref):
        jj = j - NJ
        zt = zb_ref[:, pl.ds(pl.multiple_of(jj * LT, LT), LT)]
        zs = jnp.where(zt >= thr[...], zt, 0.0)
        zs_ref[...] = zs
        part = jax.lax.dot_general(
            zs.astype(jnp.bfloat16), wd_ref[...], (((1,), (1,)), ((), ())),
            preferred_element_type=jnp.float32)

        @pl.when(j == NJ)
        def _():
            recon_ref[...] = part + bdec_ref[...]

        @pl.when(j > NJ)
        def _():
            recon_ref[...] += part

    @pl.when((i >= 1) & (j >= NJ) & (zb_cur == 1))
    def _():
        _decode(zbuf0)

    @pl.when((i >= 1) & (j >= NJ) & (zb_cur == 0))
    def _():
        _decode(zbuf1)


def _sae_call(x, b_pre2, W_enc, b_enc2, W_dec, b_dec2, interpret=False):
    return pl.pallas_call(
        _body,
        grid=(NI + 1, 2 * NJ),
        in_specs=[
            pl.BlockSpec((RT, D_MODEL), lambda i, j: (jnp.minimum(i, NI - 1), 0)),
            pl.BlockSpec((1, D_MODEL), lambda i, j: (0, 0)),
            pl.BlockSpec((LT, D_MODEL),
                         lambda i, j: (jnp.where(i >= NI, NJ - 1,
                                                 jnp.minimum(j, NJ - 1)), 0)),
            pl.BlockSpec((1, LT),
                         lambda i, j: (0, jnp.where(i >= NI, NJ - 1,
                                                    jnp.minimum(j, NJ - 1)))),
            pl.BlockSpec((D_MODEL, LT), lambda i, j: (0, jnp.maximum(j - NJ, 0))),
            pl.BlockSpec((1, D_MODEL), lambda i, j: (0, 0)),
        ],
        out_specs=[
            pl.BlockSpec((RT, D_MODEL), lambda i, j: (jnp.maximum(i - 1, 0), 0)),
            pl.BlockSpec((RT, LT),
                         lambda i, j: (jnp.maximum(i - 1, 0),
                                       jnp.where(i == 0, 0,
                                                 jnp.maximum(j - NJ, 0)))),
        ],
        out_shape=[
            jax.ShapeDtypeStruct((N_TOK, D_MODEL), jnp.float32),
            jax.ShapeDtypeStruct((N_TOK, D_SAE), jnp.float32),
        ],
        scratch_shapes=[
            pltpu.VMEM((RT, D_MODEL), jnp.float32),
            pltpu.VMEM((RT, D_SAE), jnp.float32),
            pltpu.VMEM((RT, D_SAE), jnp.float32),
            pltpu.VMEM((RT, 1), jnp.int32),
            pltpu.VMEM((RT, 1), jnp.float32),
        ],
        compiler_params=pltpu.CompilerParams(
            dimension_semantics=("arbitrary", "arbitrary"),
        ),
        interpret=interpret,
    )(x, b_pre2, W_enc, b_enc2, W_dec, b_dec2)


def kernel(x, b_pre, W_enc, b_enc, W_dec, b_dec):
    recon, zs = _sae_call(
        x,
        b_pre.reshape(1, D_MODEL),
        W_enc,
        b_enc.reshape(1, D_SAE),
        W_dec.astype(jnp.bfloat16),
        b_dec.reshape(1, D_MODEL),
    )
    return (recon, zs)


# speedup vs baseline: 9.2831x; 1.0681x over previous
import jax
import jax.numpy as jnp
import numpy as np
from jax.experimental import pallas as pl
from jax.experimental.pallas import tpu as pltpu

N_TOK = 8192
D_MODEL = 2048
D_SAE = 16384
K = 64

RT = 256
LT = 512
NJ = D_SAE // LT
NI = N_TOK // RT

_INT_MIN = np.int32(-2147483648)


def _key_to_f32(k):
    fbits = jnp.where(k < 0, k ^ _INT_MIN, ~k)
    return jax.lax.bitcast_convert_type(fbits, jnp.float32)


def _body(x_ref, bpre_ref, we_ref, benc_ref, wd_ref, bdec_ref,
          recon_ref, zs_ref, xs, zbuf0, zbuf1, tsel, thr):
    i = pl.program_id(0)
    j = pl.program_id(1)

    par = i % 2

    @pl.when(j == 0)
    def _():
        xs[...] = x_ref[...] - bpre_ref[...]

    def _encode_and_sel(zb_enc, zb_sel):
        z = jax.lax.dot_general(
            xs[...], we_ref[...], (((1,), (1,)), ((), ())),
            preferred_element_type=jnp.float32) + benc_ref[...]
        zb_enc[:, pl.ds(pl.multiple_of(j * LT, LT), LT)] = z

        t = jnp.where(j == 0, jnp.zeros((RT, 1), jnp.int32), tsel[...])
        bit = jnp.left_shift(jnp.int32(1), 31 - j)
        cand = t | bit
        cf = _key_to_f32(cand)
        cnt = jnp.sum((zb_sel[...] >= cf).astype(jnp.int32), axis=1,
                      keepdims=True)
        t = jnp.where(cnt >= K, cand, t)
        tsel[...] = t

        @pl.when(j == NJ - 1)
        def _():
            thr[...] = _key_to_f32(t)

    @pl.when((j < NJ) & (par == 0))
    def _():
        _encode_and_sel(zbuf0, zbuf1)

    @pl.when((j < NJ) & (par == 1))
    def _():
        _encode_and_sel(zbuf1, zbuf0)

    def _decode(zb_ref):
        jj = j - NJ
        zt = zb_ref[:, pl.ds(pl.multiple_of(jj * LT, LT), LT)]
        zs = jnp.where(zt >= thr[...], zt, 0.0)
        zs_ref[...] = zs
        part = jax.lax.dot_general(
            zs.astype(jnp.bfloat16), wd_ref[...], (((1,), (1,)), ((), ())),
            preferred_element_type=jnp.float32)

        @pl.when(j == NJ)
        def _():
            recon_ref[...] = part + bdec_ref[...]

        @pl.when(j > NJ)
        def _():
            recon_ref[...] += part

    @pl.when((j >= NJ) & (par == 1))
    def _():
        _decode(zbuf0)

    @pl.when((j >= NJ) & (par == 0))
    def _():
        _decode(zbuf1)


def _sae_call(x, b_pre2, W_enc, b_enc2, W_dec, b_dec2, interpret=False):
    return pl.pallas_call(
        _body,
        grid=(NI + 1, 2 * NJ),
        in_specs=[
            pl.BlockSpec((RT, D_MODEL), lambda i, j: (jnp.minimum(i, NI - 1), 0)),
            pl.BlockSpec((1, D_MODEL), lambda i, j: (0, 0)),
            pl.BlockSpec((LT, D_MODEL),
                         lambda i, j: (jnp.where(i >= NI, NJ - 1,
                                                 jnp.minimum(j, NJ - 1)), 0)),
            pl.BlockSpec((1, LT),
                         lambda i, j: (0, jnp.where(i >= NI, NJ - 1,
                                                    jnp.minimum(j, NJ - 1)))),
            pl.BlockSpec((D_MODEL, LT), lambda i, j: (0, jnp.maximum(j - NJ, 0))),
            pl.BlockSpec((1, D_MODEL), lambda i, j: (0, 0)),
        ],
        out_specs=[
            pl.BlockSpec((RT, D_MODEL), lambda i, j: (jnp.maximum(i - 1, 0), 0)),
            pl.BlockSpec((RT, LT),
                         lambda i, j: (jnp.maximum(i - 1, 0),
                                       jnp.where(i == 0, 0,
                                                 jnp.maximum(j - NJ, 0)))),
        ],
        out_shape=[
            jax.ShapeDtypeStruct((N_TOK, D_MODEL), jnp.float32),
            jax.ShapeDtypeStruct((N_TOK, D_SAE), jnp.float32),
        ],
        scratch_shapes=[
            pltpu.VMEM((RT, D_MODEL), jnp.float32),
            pltpu.VMEM((RT, D_SAE), jnp.float32),
            pltpu.VMEM((RT, D_SAE), jnp.float32),
            pltpu.VMEM((RT, 1), jnp.int32),
            pltpu.VMEM((RT, 1), jnp.float32),
        ],
        compiler_params=pltpu.CompilerParams(
            dimension_semantics=("arbitrary", "arbitrary"),
        ),
        interpret=interpret,
    )(x, b_pre2, W_enc, b_enc2, W_dec, b_dec2)


def kernel(x, b_pre, W_enc, b_enc, W_dec, b_dec):
    recon, zs = _sae_call(
        x,
        b_pre.reshape(1, D_MODEL),
        W_enc,
        b_enc.reshape(1, D_SAE),
        W_dec.astype(jnp.bfloat16),
        b_dec.reshape(1, D_MODEL),
    )
    return (recon, zs)


# speedup vs baseline: 9.7533x; 1.0506x over previous
import jax
import jax.numpy as jnp
import numpy as np
from jax.experimental import pallas as pl
from jax.experimental.pallas import tpu as pltpu

N_TOK = 8192
D_MODEL = 2048
D_SAE = 16384
K = 64

RT = 256
LT = 512
NJ = D_SAE // LT
NI = N_TOK // RT

_INT_MIN = np.int32(-2147483648)


def _key_to_f32(k):
    fbits = jnp.where(k < 0, k ^ _INT_MIN, ~k)
    return jax.lax.bitcast_convert_type(fbits, jnp.float32)


def _body(x_ref, bpre_ref, we_ref, benc_ref, wd_ref, bdec_ref,
          recon_ref, zs_ref, xs, zbuf0, zbuf1, tsel, thr):
    i = pl.program_id(0)
    j = pl.program_id(1)

    par = i % 2

    @pl.when(j == 0)
    def _():
        xs[...] = x_ref[...] - bpre_ref[...]

    HR = RT // 2
    rows = pl.ds(pl.multiple_of((j % 2) * HR, HR), HR)

    def _sel_bit(zb_sel, bitidx, reset):
        t = jnp.where(reset, jnp.zeros((HR, 1), jnp.int32), tsel[rows])
        cand = t | jnp.left_shift(jnp.int32(1), bitidx)
        cf = _key_to_f32(cand)
        cnt = jnp.sum((zb_sel[rows, :] >= cf).astype(jnp.int32), axis=1,
                      keepdims=True)
        tsel[rows] = jnp.where(cnt >= K, cand, t)

    def _encode_and_sel(zb_enc, zb_sel):
        z = jax.lax.dot_general(
            xs[...], we_ref[...], (((1,), (1,)), ((), ())),
            preferred_element_type=jnp.float32) + benc_ref[...]
        zb_enc[:, pl.ds(pl.multiple_of(j * LT, LT), LT)] = z
        _sel_bit(zb_sel, 15 - j // 2, False)

        @pl.when(j == NJ - 1)
        def _():
            thr[...] = _key_to_f32(tsel[...])

    @pl.when((j < NJ) & (par == 0))
    def _():
        _encode_and_sel(zbuf0, zbuf1)

    @pl.when((j < NJ) & (par == 1))
    def _():
        _encode_and_sel(zbuf1, zbuf0)

    def _decode_and_sel(zb_dec, zb_sel):
        jj = j - NJ
        zt = zb_dec[:, pl.ds(pl.multiple_of(jj * LT, LT), LT)]
        zs = jnp.where(zt >= thr[...], zt, 0.0)
        zs_ref[...] = zs
        part = jax.lax.dot_general(
            zs.astype(jnp.bfloat16), wd_ref[...], (((1,), (1,)), ((), ())),
            preferred_element_type=jnp.float32)
        _sel_bit(zb_sel, 31 - jj // 2, jj < 2)

        @pl.when(j == NJ)
        def _():
            recon_ref[...] = part + bdec_ref[...]

        @pl.when(j > NJ)
        def _():
            recon_ref[...] += part

    @pl.when((j >= NJ) & (par == 1))
    def _():
        _decode_and_sel(zbuf0, zbuf1)

    @pl.when((j >= NJ) & (par == 0))
    def _():
        _decode_and_sel(zbuf1, zbuf0)


def _sae_call(x, b_pre2, W_enc, b_enc2, W_dec, b_dec2, interpret=False):
    return pl.pallas_call(
        _body,
        grid=(NI + 1, 2 * NJ),
        in_specs=[
            pl.BlockSpec((RT, D_MODEL), lambda i, j: (jnp.minimum(i, NI - 1), 0)),
            pl.BlockSpec((1, D_MODEL), lambda i, j: (0, 0)),
            pl.BlockSpec((LT, D_MODEL),
                         lambda i, j: (jnp.where(i >= NI, NJ - 1,
                                                 jnp.minimum(j, NJ - 1)), 0)),
            pl.BlockSpec((1, LT),
                         lambda i, j: (0, jnp.where(i >= NI, NJ - 1,
                                                    jnp.minimum(j, NJ - 1)))),
            pl.BlockSpec((D_MODEL, LT), lambda i, j: (0, jnp.maximum(j - NJ, 0))),
            pl.BlockSpec((1, D_MODEL), lambda i, j: (0, 0)),
        ],
        out_specs=[
            pl.BlockSpec((RT, D_MODEL), lambda i, j: (jnp.maximum(i - 1, 0), 0)),
            pl.BlockSpec((RT, LT),
                         lambda i, j: (jnp.maximum(i - 1, 0),
                                       jnp.where(i == 0, 0,
                                                 jnp.maximum(j - NJ, 0)))),
        ],
        out_shape=[
            jax.ShapeDtypeStruct((N_TOK, D_MODEL), jnp.float32),
            jax.ShapeDtypeStruct((N_TOK, D_SAE), jnp.float32),
        ],
        scratch_shapes=[
            pltpu.VMEM((RT, D_MODEL), jnp.float32),
            pltpu.VMEM((RT, D_SAE), jnp.float32),
            pltpu.VMEM((RT, D_SAE), jnp.float32),
            pltpu.VMEM((RT, 1), jnp.int32),
            pltpu.VMEM((RT, 1), jnp.float32),
        ],
        compiler_params=pltpu.CompilerParams(
            dimension_semantics=("arbitrary", "arbitrary"),
        ),
        interpret=interpret,
    )(x, b_pre2, W_enc, b_enc2, W_dec, b_dec2)


def kernel(x, b_pre, W_enc, b_enc, W_dec, b_dec):
    recon, zs = _sae_call(
        x,
        b_pre.reshape(1, D_MODEL),
        W_enc,
        b_enc.reshape(1, D_SAE),
        W_dec.astype(jnp.bfloat16),
        b_dec.reshape(1, D_MODEL),
    )
    return (recon, zs)
